# baseline (device time: 75041 ns/iter reference)
import os

import jax
import jax.numpy as jnp
from jax import lax
from jax.experimental import pallas as pl
from jax.experimental.pallas import tpu as pltpu

ABLATE = int(os.environ.get("ABLATE", "0"))
DO_RS = ABLATE != 2
DO_AG = ABLATE not in (1, 2)
DO_COMPUTE = ABLATE != 4

N_DEV = 16
B, S, C_IN, C_OUT = 4, 1024, 512, 512
ROWS = B * S
HALF = ROWS // 2
N8 = 8
HOPS8 = N8 - 1
CHUNK8 = HALF // N8
SUB = CHUNK8 // 2

RING_A = [0, 4, 8, 12, 15, 11, 7, 3]
RING_B = [1, 5, 9, 13, 14, 10, 6, 2]
Q = [0] * N_DEV
RIGHT8 = [0] * N_DEV
LEFT8 = [0] * N_DEV
PARTNER = [0] * N_DEV
for _i in range(N8):
    _a, _b = RING_A[_i], RING_B[_i]
    Q[_a] = Q[_b] = _i
    RIGHT8[_a], LEFT8[_a] = RING_A[(_i + 1) % N8], RING_A[(_i - 1) % N8]
    RIGHT8[_b], LEFT8[_b] = RING_B[(_i + 1) % N8], RING_B[(_i - 1) % N8]
    PARTNER[_a], PARTNER[_b] = _b, _a


def _lut(table, idx):
    acc = jnp.int32(table[0])
    for i in range(1, len(table)):
        acc = jnp.where(idx == i, jnp.int32(table[i]), acc)
    return acc


def kernel(x, k, Wp):
    def body(x_ref, k_ref, w_ref, out_ref,
             cw_stage, ccw_stage, cw_exst, ccw_exst,
             cw_ag, ccw_ag,
             cw1_rs_send, cw1_rs_recv, cw2_rs_send, cw2_rs_recv,
             ccw1_rs_send, ccw1_rs_recv, ccw2_rs_send, ccw2_rs_recv,
             cw1_ag_send, cw1_ag_recv, cw2_ag_send, cw2_ag_recv,
             ccw1_ag_send, ccw1_ag_recv, ccw2_ag_send, ccw2_ag_recv,
             ex_send, ex_recv):
        my = lax.axis_index("i")
        q = _lut(Q, my)
        right = _lut(RIGHT8, my)
        left = _lut(LEFT8, my)
        partner = _lut(PARTNER, my)

        barrier = pltpu.get_barrier_semaphore()
        for nbr in (left, right, partner):
            pl.semaphore_signal(barrier, inc=1, device_id=(nbr,),
                                device_id_type=pl.DeviceIdType.MESH)
        pl.semaphore_wait(barrier, 3)

        kv = k_ref[:, :]
        wv_bf = w_ref[:, :].astype(jnp.bfloat16)

        def compute_c8(c8, b_base, buf):
            b = lax.div(c8, S // CHUNK8) + b_base
            s0 = pl.multiple_of(lax.rem(c8, S // CHUNK8) * CHUNK8, CHUNK8)
            xc = x_ref[b, pl.ds(s0, CHUNK8), :]
            hs = pl.multiple_of(jnp.maximum(s0 - 8, 0), 8)
            halo = x_ref[b, pl.ds(hs, 8), :][5:8]
            halo = jnp.where(s0 == 0, jnp.zeros_like(halo), halo)
            xe = jnp.concatenate([halo, xc], axis=0)
            accv = xe[3:3 + CHUNK8] * kv[3][None, :]
            for t in range(3):
                accv = accv + xe[t:t + CHUNK8] * kv[t][None, :]
            av = accv / (1.0 + jnp.exp(-accv))
            buf[pl.ds(c8 * CHUNK8, CHUNK8), :] = (
                jax.lax.dot_general(
                    av.astype(jnp.bfloat16), wv_bf, (((1,), (0,)), ((), ())),
                    preferred_element_type=jnp.float32,
                ).astype(jnp.bfloat16)
            )

        def chunk8(i):
            return lax.rem(q + i + 2 * N8, N8)

        def rd(src, dst, send_sem, recv_sem, dev):
            return pltpu.make_async_remote_copy(
                src_ref=src, dst_ref=dst, send_sem=send_sem,
                recv_sem=recv_sem, device_id=(dev,),
                device_id_type=pl.DeviceIdType.MESH,
            )

        if DO_COMPUTE:
            compute_c8(chunk8(0), 0, cw_ag)
            compute_c8(chunk8(0), 2, ccw_ag)

        pending_sends = []

        rs_sems = [(cw1_rs_send, cw1_rs_recv), (cw2_rs_send, cw2_rs_recv),
                   (ccw1_rs_send, ccw1_rs_recv), (ccw2_rs_send, ccw2_rs_recv)]

        cwx = [None] * 2
        ccwx = [None] * 2
        for s in range(HOPS8):
            if DO_RS:
                o_cw = chunk8(-s) * CHUNK8
                o_ccw = chunk8(s) * CHUNK8
                rs = [rd(cw_ag.at[pl.ds(o_cw + u * SUB, SUB), :],
                         cw_stage.at[s, pl.ds(u * SUB, SUB), :],
                         rs_sems[u][0].at[s], rs_sems[u][1].at[s], right)
                      for u in range(2)]
                rs += [rd(ccw_ag.at[pl.ds(o_ccw + u * SUB, SUB), :],
                          ccw_stage.at[s, pl.ds(u * SUB, SUB), :],
                          rs_sems[2 + u][0].at[s], rs_sems[2 + u][1].at[s],
                          left)
                       for u in range(2)]
                for r in rs:
                    r.start()
                pending_sends += rs
            if DO_COMPUTE:
                compute_c8(chunk8(-s - 1), 0, cw_ag)
                compute_c8(chunk8(s + 1), 2, ccw_ag)
            if not DO_RS:
                continue
            off_cw = chunk8(-s - 1) * CHUNK8
            off_ccw = chunk8(s + 1) * CHUNK8
            for u in range(2):
                rs[u].wait_recv()
                o = off_cw + u * SUB
                summed = (cw_ag[pl.ds(o, SUB), :].astype(jnp.float32)
                          + cw_stage[s, pl.ds(u * SUB, SUB), :]
                          .astype(jnp.float32))
                cw_ag[pl.ds(o, SUB), :] = summed.astype(jnp.bfloat16)
                if s == HOPS8 - 1:
                    out_ref[pl.ds(o, SUB), :] = summed
                    cwx[u] = rd(cw_ag.at[pl.ds(o, SUB), :],
                                cw_exst.at[pl.ds(u * SUB, SUB), :],
                                ex_send.at[u], ex_recv.at[u], partner)
                    cwx[u].start()
            for u in range(2):
                rs[2 + u].wait_recv()
                o = off_ccw + u * SUB
                summed = (ccw_ag[pl.ds(o, SUB), :].astype(jnp.float32)
                          + ccw_stage[s, pl.ds(u * SUB, SUB), :]
                          .astype(jnp.float32))
                ccw_ag[pl.ds(o, SUB), :] = summed.astype(jnp.bfloat16)
                if s == HOPS8 - 1:
                    out_ref[pl.ds(HALF + o, SUB), :] = summed
                    ccwx[u] = rd(ccw_ag.at[pl.ds(o, SUB), :],
                                 ccw_exst.at[pl.ds(u * SUB, SUB), :],
                                 ex_send.at[2 + u], ex_recv.at[2 + u],
                                 partner)
                    ccwx[u].start()

        ag_sems = [(cw1_ag_send, cw1_ag_recv), (cw2_ag_send, cw2_ag_recv),
                   (ccw1_ag_send, ccw1_ag_recv), (ccw2_ag_send, ccw2_ag_recv)]

        def ag_rd(sysid, buf, base_off, s, dev):
            off = base_off + (sysid % 2) * SUB
            send_sems, recv_sems = ag_sems[sysid]
            return rd(buf.at[pl.ds(off, SUB), :],
                      buf.at[pl.ds(off, SUB), :],
                      send_sems.at[s], recv_sems.at[s], dev)

        prev = [None] * 4
        if DO_RS:
            off_cw = chunk8(1) * CHUNK8
            off_ccw = chunk8(-1) * CHUNK8
            for u in range(2):
                cwx[u].wait_recv()
                cwx[u].wait_send()
                o = off_cw + u * SUB
                fin = (out_ref[pl.ds(o, SUB), :]
                       + cw_exst[pl.ds(u * SUB, SUB), :].astype(jnp.float32))
                out_ref[pl.ds(o, SUB), :] = fin
                cw_ag[pl.ds(o, SUB), :] = fin.astype(jnp.bfloat16)
                if DO_AG:
                    prev[u] = ag_rd(u, cw_ag, off_cw, 0, right)
                    prev[u].start()
            for u in range(2):
                ccwx[u].wait_recv()
                ccwx[u].wait_send()
                o = off_ccw + u * SUB
                fin = (out_ref[pl.ds(HALF + o, SUB), :]
                       + ccw_exst[pl.ds(u * SUB, SUB), :].astype(jnp.float32))
                out_ref[pl.ds(HALF + o, SUB), :] = fin
                ccw_ag[pl.ds(o, SUB), :] = fin.astype(jnp.bfloat16)
                if DO_AG:
                    prev[2 + u] = ag_rd(2 + u, ccw_ag, off_ccw, 0, left)
                    prev[2 + u].start()
            if DO_AG:
                pending_sends += prev

        for s in range(1, HOPS8 if DO_AG else 1):
            cw_off = chunk8(1 - s) * CHUNK8
            ccw_off = chunk8(s - 1) * CHUNK8
            for i, (buf, base, dev) in enumerate(
                    [(cw_ag, cw_off, right), (cw_ag, cw_off, right),
                     (ccw_ag, ccw_off, left), (ccw_ag, ccw_off, left)]):
                prev[i].wait_recv()
                prev[i] = ag_rd(i, buf, base, s, dev)
                prev[i].start()
            pending_sends += prev
            out_ref[pl.ds(cw_off, CHUNK8), :] = (
                cw_ag[pl.ds(cw_off, CHUNK8), :].astype(jnp.float32))
            out_ref[pl.ds(HALF + ccw_off, CHUNK8), :] = (
                ccw_ag[pl.ds(ccw_off, CHUNK8), :].astype(jnp.float32))
        if DO_AG:
            for r in prev:
                r.wait_recv()
            o = chunk8(-HOPS8 + 1) * CHUNK8
            out_ref[pl.ds(o, CHUNK8), :] = (
                cw_ag[pl.ds(o, CHUNK8), :].astype(jnp.float32))
            o = chunk8(HOPS8 - 1) * CHUNK8
            out_ref[pl.ds(HALF + o, CHUNK8), :] = (
                ccw_ag[pl.ds(o, CHUNK8), :].astype(jnp.float32))

        for r in pending_sends:
            r.wait_send()

    out = pl.pallas_call(
        body,
        out_shape=jax.ShapeDtypeStruct((ROWS, C_OUT), jnp.float32),
        in_specs=[pl.BlockSpec(memory_space=pltpu.VMEM)] * 3,
        out_specs=pl.BlockSpec(memory_space=pltpu.VMEM),
        scratch_shapes=[
            pltpu.VMEM((HOPS8, CHUNK8, C_OUT), jnp.bfloat16),
            pltpu.VMEM((HOPS8, CHUNK8, C_OUT), jnp.bfloat16),
            pltpu.VMEM((CHUNK8, C_OUT), jnp.bfloat16),
            pltpu.VMEM((CHUNK8, C_OUT), jnp.bfloat16),
            pltpu.VMEM((HALF, C_OUT), jnp.bfloat16),
            pltpu.VMEM((HALF, C_OUT), jnp.bfloat16),
        ] + [pltpu.SemaphoreType.DMA((HOPS8,))] * 16
          + [pltpu.SemaphoreType.DMA((4,))] * 2,
        compiler_params=pltpu.CompilerParams(collective_id=0),
    )(x, k, Wp)
    return out.reshape(B, S, C_OUT)


# device time: 74585 ns/iter; 1.0061x vs baseline; 1.0061x over previous
import os

import jax
import jax.numpy as jnp
from jax import lax
from jax.experimental import pallas as pl
from jax.experimental.pallas import tpu as pltpu

ABLATE = int(os.environ.get("ABLATE", "0"))
DO_RS = ABLATE != 2
DO_AG = ABLATE not in (1, 2)
DO_COMPUTE = ABLATE != 4

N_DEV = 16
B, S, C_IN, C_OUT = 4, 1024, 512, 512
ROWS = B * S
HALF = ROWS // 2
N8 = 8
HOPS8 = N8 - 1
CHUNK8 = HALF // N8
SUB = CHUNK8 // 2

RING_A = [0, 4, 8, 12, 15, 11, 7, 3]
RING_B = [1, 5, 9, 13, 14, 10, 6, 2]
Q = [0] * N_DEV
RIGHT8 = [0] * N_DEV
LEFT8 = [0] * N_DEV
PARTNER = [0] * N_DEV
for _i in range(N8):
    _a, _b = RING_A[_i], RING_B[_i]
    Q[_a] = Q[_b] = _i
    RIGHT8[_a], LEFT8[_a] = RING_A[(_i + 1) % N8], RING_A[(_i - 1) % N8]
    RIGHT8[_b], LEFT8[_b] = RING_B[(_i + 1) % N8], RING_B[(_i - 1) % N8]
    PARTNER[_a], PARTNER[_b] = _b, _a


def _lut(table, idx):
    acc = jnp.int32(table[0])
    for i in range(1, len(table)):
        acc = jnp.where(idx == i, jnp.int32(table[i]), acc)
    return acc


def kernel(x, k, Wp):
    def body(x_ref, k_ref, w_ref, out_ref,
             cw_stage, ccw_stage, cw_exst, ccw_exst,
             cw_ag, ccw_ag,
             cw1_rs_send, cw1_rs_recv, cw2_rs_send, cw2_rs_recv,
             ccw1_rs_send, ccw1_rs_recv, ccw2_rs_send, ccw2_rs_recv,
             cw1_ag_send, cw1_ag_recv, cw2_ag_send, cw2_ag_recv,
             ccw1_ag_send, ccw1_ag_recv, ccw2_ag_send, ccw2_ag_recv,
             ex_send, ex_recv):
        my = lax.axis_index("i")
        q = _lut(Q, my)
        right = _lut(RIGHT8, my)
        left = _lut(LEFT8, my)
        partner = _lut(PARTNER, my)

        barrier = pltpu.get_barrier_semaphore()
        for nbr in (left, right, partner):
            pl.semaphore_signal(barrier, inc=1, device_id=(nbr,),
                                device_id_type=pl.DeviceIdType.MESH)
        pl.semaphore_wait(barrier, 3)

        kv = k_ref[:, :]
        wv_bf = w_ref[:, :].astype(jnp.bfloat16)

        def compute_c8(c8, b_base, buf):
            b = lax.div(c8, S // CHUNK8) + b_base
            s0 = pl.multiple_of(lax.rem(c8, S // CHUNK8) * CHUNK8, CHUNK8)
            xc = x_ref[b, pl.ds(s0, CHUNK8), :]
            hs = pl.multiple_of(jnp.maximum(s0 - 8, 0), 8)
            halo = x_ref[b, pl.ds(hs, 8), :][5:8]
            halo = jnp.where(s0 == 0, jnp.zeros_like(halo), halo)
            xe = jnp.concatenate([halo, xc], axis=0)
            accv = xe[3:3 + CHUNK8] * kv[3][None, :]
            for t in range(3):
                accv = accv + xe[t:t + CHUNK8] * kv[t][None, :]
            av = accv / (1.0 + jnp.exp(-accv))
            buf[pl.ds(c8 * CHUNK8, CHUNK8), :] = (
                jax.lax.dot_general(
                    av.astype(jnp.bfloat16), wv_bf, (((1,), (0,)), ((), ())),
                    preferred_element_type=jnp.float32,
                ).astype(jnp.bfloat16)
            )

        def chunk8(i):
            return lax.rem(q + i + 2 * N8, N8)

        def rd(src, dst, send_sem, recv_sem, dev):
            return pltpu.make_async_remote_copy(
                src_ref=src, dst_ref=dst, send_sem=send_sem,
                recv_sem=recv_sem, device_id=(dev,),
                device_id_type=pl.DeviceIdType.MESH,
            )

        if DO_COMPUTE:
            compute_c8(chunk8(0), 0, cw_ag)
            compute_c8(chunk8(0), 2, ccw_ag)

        pending_sends = []

        rs_sems = [(cw1_rs_send, cw1_rs_recv), (cw2_rs_send, cw2_rs_recv),
                   (ccw1_rs_send, ccw1_rs_recv), (ccw2_rs_send, ccw2_rs_recv)]

        cwx = [None] * 2
        ccwx = [None] * 2
        for s in range(HOPS8):
            if DO_RS:
                o_cw = chunk8(-s) * CHUNK8
                o_ccw = chunk8(s) * CHUNK8
                rs = [rd(cw_ag.at[pl.ds(o_cw + u * SUB, SUB), :],
                         cw_stage.at[s, pl.ds(u * SUB, SUB), :],
                         rs_sems[u][0].at[s], rs_sems[u][1].at[s], right)
                      for u in range(2)]
                rs += [rd(ccw_ag.at[pl.ds(o_ccw + u * SUB, SUB), :],
                          ccw_stage.at[s, pl.ds(u * SUB, SUB), :],
                          rs_sems[2 + u][0].at[s], rs_sems[2 + u][1].at[s],
                          left)
                       for u in range(2)]
                for r in rs:
                    r.start()
                pending_sends += rs
            if DO_COMPUTE:
                compute_c8(chunk8(-s - 1), 0, cw_ag)
                compute_c8(chunk8(s + 1), 2, ccw_ag)
            if not DO_RS:
                continue
            off_cw = chunk8(-s - 1) * CHUNK8
            off_ccw = chunk8(s + 1) * CHUNK8
            for u in range(2):
                rs[u].wait_recv()
                o = off_cw + u * SUB
                summed = (cw_ag[pl.ds(o, SUB), :]
                          + cw_stage[s, pl.ds(u * SUB, SUB), :])
                cw_ag[pl.ds(o, SUB), :] = summed
                if s == HOPS8 - 1:
                    out_ref[pl.ds(o, SUB), :] = summed.astype(jnp.float32)
                    cwx[u] = rd(cw_ag.at[pl.ds(o, SUB), :],
                                cw_exst.at[pl.ds(u * SUB, SUB), :],
                                ex_send.at[u], ex_recv.at[u], partner)
                    cwx[u].start()
            for u in range(2):
                rs[2 + u].wait_recv()
                o = off_ccw + u * SUB
                summed = (ccw_ag[pl.ds(o, SUB), :]
                          + ccw_stage[s, pl.ds(u * SUB, SUB), :])
                ccw_ag[pl.ds(o, SUB), :] = summed
                if s == HOPS8 - 1:
                    out_ref[pl.ds(HALF + o, SUB), :] = (
                        summed.astype(jnp.float32))
                    ccwx[u] = rd(ccw_ag.at[pl.ds(o, SUB), :],
                                 ccw_exst.at[pl.ds(u * SUB, SUB), :],
                                 ex_send.at[2 + u], ex_recv.at[2 + u],
                                 partner)
                    ccwx[u].start()

        ag_sems = [(cw1_ag_send, cw1_ag_recv), (cw2_ag_send, cw2_ag_recv),
                   (ccw1_ag_send, ccw1_ag_recv), (ccw2_ag_send, ccw2_ag_recv)]

        def ag_rd(sysid, buf, base_off, s, dev):
            off = base_off + (sysid % 2) * SUB
            send_sems, recv_sems = ag_sems[sysid]
            return rd(buf.at[pl.ds(off, SUB), :],
                      buf.at[pl.ds(off, SUB), :],
                      send_sems.at[s], recv_sems.at[s], dev)

        prev = [None] * 4
        if DO_RS:
            off_cw = chunk8(1) * CHUNK8
            off_ccw = chunk8(-1) * CHUNK8
            for u in range(2):
                cwx[u].wait_recv()
                cwx[u].wait_send()
                o = off_cw + u * SUB
                fin = (out_ref[pl.ds(o, SUB), :]
                       + cw_exst[pl.ds(u * SUB, SUB), :].astype(jnp.float32))
                out_ref[pl.ds(o, SUB), :] = fin
                cw_ag[pl.ds(o, SUB), :] = fin.astype(jnp.bfloat16)
                if DO_AG:
                    prev[u] = ag_rd(u, cw_ag, off_cw, 0, right)
                    prev[u].start()
            for u in range(2):
                ccwx[u].wait_recv()
                ccwx[u].wait_send()
                o = off_ccw + u * SUB
                fin = (out_ref[pl.ds(HALF + o, SUB), :]
                       + ccw_exst[pl.ds(u * SUB, SUB), :].astype(jnp.float32))
                out_ref[pl.ds(HALF + o, SUB), :] = fin
                ccw_ag[pl.ds(o, SUB), :] = fin.astype(jnp.bfloat16)
                if DO_AG:
                    prev[2 + u] = ag_rd(2 + u, ccw_ag, off_ccw, 0, left)
                    prev[2 + u].start()
            if DO_AG:
                pending_sends += prev

        for s in range(1, HOPS8 if DO_AG else 1):
            cw_off = chunk8(1 - s) * CHUNK8
            ccw_off = chunk8(s - 1) * CHUNK8
            for i, (buf, base, dev) in enumerate(
                    [(cw_ag, cw_off, right), (cw_ag, cw_off, right),
                     (ccw_ag, ccw_off, left), (ccw_ag, ccw_off, left)]):
                prev[i].wait_recv()
                prev[i] = ag_rd(i, buf, base, s, dev)
                prev[i].start()
            pending_sends += prev
            out_ref[pl.ds(cw_off, CHUNK8), :] = (
                cw_ag[pl.ds(cw_off, CHUNK8), :].astype(jnp.float32))
            out_ref[pl.ds(HALF + ccw_off, CHUNK8), :] = (
                ccw_ag[pl.ds(ccw_off, CHUNK8), :].astype(jnp.float32))
        if DO_AG:
            for r in prev:
                r.wait_recv()
            o = chunk8(-HOPS8 + 1) * CHUNK8
            out_ref[pl.ds(o, CHUNK8), :] = (
                cw_ag[pl.ds(o, CHUNK8), :].astype(jnp.float32))
            o = chunk8(HOPS8 - 1) * CHUNK8
            out_ref[pl.ds(HALF + o, CHUNK8), :] = (
                ccw_ag[pl.ds(o, CHUNK8), :].astype(jnp.float32))

        for r in pending_sends:
            r.wait_send()

    out = pl.pallas_call(
        body,
        out_shape=jax.ShapeDtypeStruct((ROWS, C_OUT), jnp.float32),
        in_specs=[pl.BlockSpec(memory_space=pltpu.VMEM)] * 3,
        out_specs=pl.BlockSpec(memory_space=pltpu.VMEM),
        scratch_shapes=[
            pltpu.VMEM((HOPS8, CHUNK8, C_OUT), jnp.bfloat16),
            pltpu.VMEM((HOPS8, CHUNK8, C_OUT), jnp.bfloat16),
            pltpu.VMEM((CHUNK8, C_OUT), jnp.bfloat16),
            pltpu.VMEM((CHUNK8, C_OUT), jnp.bfloat16),
            pltpu.VMEM((HALF, C_OUT), jnp.bfloat16),
            pltpu.VMEM((HALF, C_OUT), jnp.bfloat16),
        ] + [pltpu.SemaphoreType.DMA((HOPS8,))] * 16
          + [pltpu.SemaphoreType.DMA((4,))] * 2,
        compiler_params=pltpu.CompilerParams(collective_id=0),
    )(x, k, Wp)
    return out.reshape(B, S, C_OUT)
